# baseline (speedup 1.0000x reference)
import jax
import jax.numpy as jnp
from jax.experimental import pallas as pl
from jax.experimental.pallas import tpu as pltpu

N = 4096
D_IN = 64
D_OUT = 64
K = 2
BN = 256
BM = 2048
MJ = N // BM


def _body(x_ref, adj_ref, w_ref, out_ref, h_ref):
    i = pl.program_id(0)
    j = pl.program_id(1)

    @pl.when((i == 0) & (j == 0))
    def _():
        h_ref[...] = jnp.dot(x_ref[...], w_ref[...],
                             preferred_element_type=jnp.float32).astype(
                                 jnp.bfloat16)

    a = (adj_ref[0] + adj_ref[1]).astype(jnp.bfloat16)
    part = jnp.dot(a, h_ref[pl.ds(j * BM, BM), :],
                   preferred_element_type=jnp.float32)

    @pl.when(j == 0)
    def _():
        out_ref[...] = part

    @pl.when(j == MJ - 1)
    def _():
        out_ref[...] = jnp.maximum(out_ref[...] + part, 0.0)


@jax.jit
def kernel(input, adj_list, W):
    return pl.pallas_call(
        _body,
        grid=(N // BN, MJ),
        in_specs=[
            pl.BlockSpec((N, D_IN), lambda i, j: (0, 0)),
            pl.BlockSpec((K, BN, BM), lambda i, j: (0, i, j)),
            pl.BlockSpec((D_IN, D_OUT), lambda i, j: (0, 0)),
        ],
        out_specs=pl.BlockSpec((BN, D_OUT), lambda i, j: (i, 0)),
        out_shape=jax.ShapeDtypeStruct((N, D_OUT), jnp.float32),
        scratch_shapes=[pltpu.VMEM((N, D_OUT), jnp.bfloat16)],
    )(input, adj_list, W)


# speedup vs baseline: 1.0257x; 1.0257x over previous
import jax
import jax.numpy as jnp
from jax import lax
from jax.experimental import pallas as pl
from jax.experimental.pallas import tpu as pltpu

N = 4096
D_IN = 64
D_OUT = 64
K = 2
BN = 256
NBUF = 3
NB = N // BN


def _body(x_ref, adj_ref, w_ref, out_ref, ring_ref, sem, h_ref):
    h_ref[...] = jnp.dot(x_ref[...], w_ref[...],
                         preferred_element_type=jnp.float32).astype(
                             jnp.bfloat16)

    def dma(i, slot, k):
        return pltpu.make_async_copy(
            adj_ref.at[pl.ds(k * N + i * BN, BN)],
            ring_ref.at[slot, k],
            sem.at[slot, k])

    for s in range(NBUF):
        dma(s, s, 0).start()
        dma(s, s, 1).start()

    def step(i, _):
        slot = lax.rem(i, NBUF)
        dma(i, slot, 0).wait()
        dma(i, slot, 1).wait()
        a = (ring_ref[slot, 0] + ring_ref[slot, 1]).astype(jnp.bfloat16)
        part = jnp.dot(a, h_ref[...], preferred_element_type=jnp.float32)
        out_ref[pl.ds(i * BN, BN), :] = jnp.maximum(part, 0.0)

        @pl.when(i + NBUF < NB)
        def _():
            dma(i + NBUF, slot, 0).start()
            dma(i + NBUF, slot, 1).start()

        return 0

    lax.fori_loop(0, NB, step, 0)


@jax.jit
def kernel(input, adj_list, W):
    adj_flat = adj_list.reshape(K * N, N)
    return pl.pallas_call(
        _body,
        in_specs=[
            pl.BlockSpec(memory_space=pltpu.VMEM),
            pl.BlockSpec(memory_space=pl.ANY),
            pl.BlockSpec(memory_space=pltpu.VMEM),
        ],
        out_specs=pl.BlockSpec(memory_space=pltpu.VMEM),
        out_shape=jax.ShapeDtypeStruct((N, D_OUT), jnp.float32),
        scratch_shapes=[
            pltpu.VMEM((NBUF, K, BN, N), jnp.float32),
            pltpu.SemaphoreType.DMA((NBUF, K)),
            pltpu.VMEM((N, D_OUT), jnp.bfloat16),
        ],
    )(input, adj_flat, W)


# speedup vs baseline: 1.1157x; 1.0877x over previous
import jax
import jax.numpy as jnp
from jax.experimental import pallas as pl
from jax.experimental.pallas import tpu as pltpu

N = 4096
D_IN = 64
D_OUT = 64
K = 2
BN = 256


def _body(x_ref, adj_ref, w_ref, out_ref, h_ref):
    @pl.when(pl.program_id(0) == 0)
    def _():
        h_ref[...] = jnp.dot(x_ref[...], w_ref[...],
                             preferred_element_type=jnp.float32).astype(
                                 jnp.bfloat16)

    a = (adj_ref[0] + adj_ref[1]).astype(jnp.bfloat16)
    acc = jnp.dot(a, h_ref[...], preferred_element_type=jnp.float32)
    out_ref[...] = jnp.maximum(acc, 0.0)


@jax.jit
def kernel(input, adj_list, W):
    return pl.pallas_call(
        _body,
        grid=(N // BN,),
        in_specs=[
            pl.BlockSpec((N, D_IN), lambda i: (0, 0)),
            pl.BlockSpec((K, BN, N), lambda i: (0, i, 0)),
            pl.BlockSpec((D_IN, D_OUT), lambda i: (0, 0)),
        ],
        out_specs=pl.BlockSpec((BN, D_OUT), lambda i: (i, 0)),
        out_shape=jax.ShapeDtypeStruct((N, D_OUT), jnp.float32),
        scratch_shapes=[pltpu.VMEM((N, D_OUT), jnp.bfloat16)],
    )(input, adj_list, W)


# speedup vs baseline: 1.1695x; 1.0482x over previous
import jax
import jax.numpy as jnp
from jax import lax
from jax.experimental import pallas as pl
from jax.experimental.pallas import tpu as pltpu

N = 4096
D_IN = 64
D_OUT = 64
K = 2
BN = 256


def _body(x_ref, adj_ref, w_ref, out_ref, ht_ref):
    @pl.when(pl.program_id(0) == 0)
    def _():
        h = jnp.dot(x_ref[...], w_ref[...],
                    preferred_element_type=jnp.float32)
        ht_ref[...] = h.T.astype(jnp.bfloat16)

    a = (adj_ref[0] + adj_ref[1]).astype(jnp.bfloat16)
    part_t = lax.dot_general(ht_ref[...], a, (((1,), (1,)), ((), ())),
                             preferred_element_type=jnp.float32)
    out_ref[...] = jnp.maximum(part_t, 0.0)


@jax.jit
def kernel(input, adj_list, W):
    out_t = pl.pallas_call(
        _body,
        grid=(N // BN,),
        in_specs=[
            pl.BlockSpec((N, D_IN), lambda i: (0, 0)),
            pl.BlockSpec((K, BN, N), lambda i: (0, i, 0)),
            pl.BlockSpec((D_IN, D_OUT), lambda i: (0, 0)),
        ],
        out_specs=pl.BlockSpec((D_OUT, BN), lambda i: (0, i)),
        out_shape=jax.ShapeDtypeStruct((D_OUT, N), jnp.float32),
        scratch_shapes=[pltpu.VMEM((D_OUT, N), jnp.bfloat16)],
    )(input, adj_list, W)
    return out_t.T
